# Initial kernel scaffold; baseline (speedup 1.0000x reference)
import jax
import jax.numpy as jnp
from jax.experimental import pallas as pl


def kernel(x, edge_index, W, b):
    raise NotImplementedError("write your pallas kernel here")


# speedup vs baseline: 12.3676x; 12.3676x over previous
import functools

import jax
import jax.numpy as jnp
from jax import lax
from jax.experimental import pallas as pl
from jax.experimental.pallas import tpu as pltpu
from jax.experimental.pallas import tpu_sc as plsc

_N = 10000
_E = 320000
_D = 128
_NC = 2
_NS = 16
_NW = _NC * _NS
_EB = 128
_NBLK = _E // _EB
_BPT = _NBLK // _NW
_EXTRA = _NBLK - _BPT * _NW
_RS = 624
_RTAIL = _N - _RS * _NS


@functools.partial(
    pl.kernel,
    out_type=jax.ShapeDtypeStruct((2 * _N, _D), jnp.float32),
    mesh=plsc.VectorSubcoreMesh(core_axis_name="c", subcore_axis_name="s"),
    scratch_types=[
        pltpu.VMEM((_EB,), jnp.int32),
        pltpu.VMEM((_EB,), jnp.int32),
        pltpu.VMEM((_EB, _D), jnp.float32),
        pltpu.VMEM_SHARED((_N, _D), jnp.float32),
    ],
)
def _sc_scatter(u_hbm, srcb_hbm, dstb_hbm, out_hbm, sidx_v, didx_v, rows_v,
                acc_sh):
    c = lax.axis_index("c")
    s = lax.axis_index("s")
    wid = c * _NS + s

    pltpu.sync_copy(u_hbm.at[pl.ds(s * _RS, _RS)],
                    acc_sh.at[pl.ds(s * _RS, _RS)])

    @pl.when(s == _NS - 1)
    def _():
        pltpu.sync_copy(u_hbm.at[pl.ds(_NS * _RS, _RTAIL)],
                        acc_sh.at[pl.ds(_NS * _RS, _RTAIL)])

    plsc.subcore_barrier()

    def _block(b):
        pltpu.sync_copy(srcb_hbm.at[pl.ds(b * _EB, _EB)], sidx_v)
        pltpu.sync_copy(dstb_hbm.at[pl.ds(b * _EB, _EB)], didx_v)
        pltpu.sync_copy(u_hbm.at[sidx_v], rows_v)
        pltpu.sync_copy(rows_v, acc_sh.at[didx_v], add=True)

    @pl.loop(0, _BPT)
    def _(i):
        _block(wid * _BPT + i)

    @pl.when(wid < _EXTRA)
    def _():
        _block(_NW * _BPT + wid)

    plsc.subcore_barrier()
    pltpu.sync_copy(acc_sh.at[pl.ds(s * _RS, _RS)],
                    out_hbm.at[pl.ds(c * _N + s * _RS, _RS)])

    @pl.when(s == _NS - 1)
    def _():
        pltpu.sync_copy(acc_sh.at[pl.ds(_NS * _RS, _RTAIL)],
                        out_hbm.at[pl.ds(c * _N + _NS * _RS, _RTAIL)])


_BN = 1000
_GN = _N // _BN


def _mm_body(x_ref, w_ref, y_ref):
    y_ref[...] = lax.dot_general(
        x_ref[...], w_ref[...], (((1,), (1,)), ((), ())),
        preferred_element_type=jnp.float32)


def _tc_matmul(x, W):
    return pl.pallas_call(
        _mm_body,
        grid=(_GN,),
        in_specs=[pl.BlockSpec((_BN, _D), lambda i: (i, 0)),
                  pl.BlockSpec((_D, _D), lambda i: (0, 0))],
        out_specs=pl.BlockSpec((_BN, _D), lambda i: (i, 0)),
        out_shape=jax.ShapeDtypeStruct((_N, _D), jnp.float32),
    )(x, W)


def _u0_body(y_ref, d0_ref, d1_ref, u0_ref, dv_ref):
    deg = d0_ref[...] + d1_ref[...] - 1.0
    dv = lax.rsqrt(deg)
    u0_ref[...] = dv * y_ref[...]
    dv_ref[...] = dv


def _tc_u0(y, degp):
    return pl.pallas_call(
        _u0_body,
        grid=(_GN,),
        in_specs=[pl.BlockSpec((_BN, _D), lambda i: (i, 0)),
                  pl.BlockSpec((_BN, _D), lambda i: (i, 0)),
                  pl.BlockSpec((_BN, _D), lambda i: (i + _GN, 0))],
        out_specs=[pl.BlockSpec((_BN, _D), lambda i: (i, 0)),
                   pl.BlockSpec((_BN, _D), lambda i: (i, 0))],
        out_shape=[jax.ShapeDtypeStruct((_N, _D), jnp.float32),
                   jax.ShapeDtypeStruct((_N, _D), jnp.float32)],
    )(y, degp, degp)


def _u1_body(p0_ref, p1_ref, u0_ref, dv_ref, u1_ref):
    dv = dv_ref[...]
    u1_ref[...] = dv * dv * (p0_ref[...] + p1_ref[...] - u0_ref[...])


def _tc_u1(p, u0, dvb):
    return pl.pallas_call(
        _u1_body,
        grid=(_GN,),
        in_specs=[pl.BlockSpec((_BN, _D), lambda i: (i, 0)),
                  pl.BlockSpec((_BN, _D), lambda i: (i + _GN, 0)),
                  pl.BlockSpec((_BN, _D), lambda i: (i, 0)),
                  pl.BlockSpec((_BN, _D), lambda i: (i, 0))],
        out_specs=pl.BlockSpec((_BN, _D), lambda i: (i, 0)),
        out_shape=jax.ShapeDtypeStruct((_N, _D), jnp.float32),
    )(p, p, u0, dvb)


def _fin_body(q0_ref, q1_ref, u1_ref, dv_ref, b_ref, o_ref):
    o_ref[...] = (dv_ref[...] * (q0_ref[...] + q1_ref[...] - u1_ref[...])
                  + b_ref[...])


def _tc_final(q, u1, dvb, b2d):
    return pl.pallas_call(
        _fin_body,
        grid=(_GN,),
        in_specs=[pl.BlockSpec((_BN, _D), lambda i: (i, 0)),
                  pl.BlockSpec((_BN, _D), lambda i: (i + _GN, 0)),
                  pl.BlockSpec((_BN, _D), lambda i: (i, 0)),
                  pl.BlockSpec((_BN, _D), lambda i: (i, 0)),
                  pl.BlockSpec((1, _D), lambda i: (0, 0))],
        out_specs=pl.BlockSpec((_BN, _D), lambda i: (i, 0)),
        out_shape=jax.ShapeDtypeStruct((_N, _D), jnp.float32),
    )(q, q, u1, dvb, b2d)


def kernel(x, edge_index, W, b):
    src = edge_index[0]
    dst = edge_index[1]
    ones = jnp.ones((_N, _D), jnp.float32)
    y = _tc_matmul(x, W)
    degp = _sc_scatter(ones, src, dst)
    u0, dvb = _tc_u0(y, degp)
    p = _sc_scatter(u0, src, dst)
    u1 = _tc_u1(p, u0, dvb)
    q = _sc_scatter(u1, src, dst)
    return _tc_final(q, u1, dvb, b.reshape(1, _D))


# speedup vs baseline: 14.8895x; 1.2039x over previous
import functools

import jax
import jax.numpy as jnp
from jax import lax
from jax.experimental import pallas as pl
from jax.experimental.pallas import tpu as pltpu
from jax.experimental.pallas import tpu_sc as plsc

_N = 10000
_E = 320000
_D = 128
_NC = 2
_NS = 16
_NW = _NC * _NS
_EB = 128
_NBLK = _E // _EB
_BPT = _NBLK // _NW
_EXTRA = _NBLK - _BPT * _NW
_RS = 624
_RTAIL = _N - _RS * _NS


@functools.partial(
    pl.kernel,
    out_type=jax.ShapeDtypeStruct((2 * _N, _D), jnp.float32),
    mesh=plsc.VectorSubcoreMesh(core_axis_name="c", subcore_axis_name="s"),
    scratch_types=[
        pltpu.VMEM((_EB,), jnp.int32),
        pltpu.VMEM((_EB,), jnp.int32),
        pltpu.VMEM((_EB, _D), jnp.float32),
        pltpu.VMEM_SHARED((_N, _D), jnp.float32),
    ],
)
def _sc_scatter(u_hbm, srcb_hbm, dstb_hbm, out_hbm, sidx_v, didx_v, rows_v,
                acc_sh):
    c = lax.axis_index("c")
    s = lax.axis_index("s")
    wid = c * _NS + s

    pltpu.sync_copy(u_hbm.at[pl.ds(s * _RS, _RS)],
                    acc_sh.at[pl.ds(s * _RS, _RS)])

    @pl.when(s == _NS - 1)
    def _():
        pltpu.sync_copy(u_hbm.at[pl.ds(_NS * _RS, _RTAIL)],
                        acc_sh.at[pl.ds(_NS * _RS, _RTAIL)])

    plsc.subcore_barrier()

    def _block(b):
        pltpu.sync_copy(srcb_hbm.at[pl.ds(b * _EB, _EB)], sidx_v)
        pltpu.sync_copy(dstb_hbm.at[pl.ds(b * _EB, _EB)], didx_v)
        pltpu.sync_copy(u_hbm.at[sidx_v], rows_v)
        pltpu.sync_copy(rows_v, acc_sh.at[didx_v], add=True)

    @pl.loop(0, _BPT)
    def _(i):
        _block(wid * _BPT + i)

    @pl.when(wid < _EXTRA)
    def _():
        _block(_NW * _BPT + wid)

    plsc.subcore_barrier()
    pltpu.sync_copy(acc_sh.at[pl.ds(s * _RS, _RS)],
                    out_hbm.at[pl.ds(c * _N + s * _RS, _RS)])

    @pl.when(s == _NS - 1)
    def _():
        pltpu.sync_copy(acc_sh.at[pl.ds(_NS * _RS, _RTAIL)],
                        out_hbm.at[pl.ds(c * _N + _NS * _RS, _RTAIL)])


@functools.partial(
    pl.kernel,
    out_type=jax.ShapeDtypeStruct((2 * _N, _D), jnp.float32),
    mesh=plsc.VectorSubcoreMesh(core_axis_name="c", subcore_axis_name="s"),
    scratch_types=[
        pltpu.VMEM((_EB,), jnp.int32),
        pltpu.VMEM((_EB, _D), jnp.float32),
        pltpu.VMEM_SHARED((_N, _D), jnp.float32),
    ],
)
def _sc_degree(ones_hbm, dstb_hbm, out_hbm, didx_v, ones_v, acc_sh):
    c = lax.axis_index("c")
    s = lax.axis_index("s")
    wid = c * _NS + s

    pltpu.sync_copy(ones_hbm.at[pl.ds(0, _EB)], ones_v)
    pltpu.sync_copy(ones_hbm.at[pl.ds(s * _RS, _RS)],
                    acc_sh.at[pl.ds(s * _RS, _RS)])

    @pl.when(s == _NS - 1)
    def _():
        pltpu.sync_copy(ones_hbm.at[pl.ds(_NS * _RS, _RTAIL)],
                        acc_sh.at[pl.ds(_NS * _RS, _RTAIL)])

    plsc.subcore_barrier()

    def _block(b):
        pltpu.sync_copy(dstb_hbm.at[pl.ds(b * _EB, _EB)], didx_v)
        pltpu.sync_copy(ones_v, acc_sh.at[didx_v], add=True)

    @pl.loop(0, _BPT)
    def _(i):
        _block(wid * _BPT + i)

    @pl.when(wid < _EXTRA)
    def _():
        _block(_NW * _BPT + wid)

    plsc.subcore_barrier()
    pltpu.sync_copy(acc_sh.at[pl.ds(s * _RS, _RS)],
                    out_hbm.at[pl.ds(c * _N + s * _RS, _RS)])

    @pl.when(s == _NS - 1)
    def _():
        pltpu.sync_copy(acc_sh.at[pl.ds(_NS * _RS, _RTAIL)],
                        out_hbm.at[pl.ds(c * _N + _NS * _RS, _RTAIL)])


_BN = 1000
_GN = _N // _BN


def _mm_body(x_ref, w_ref, y_ref):
    y_ref[...] = lax.dot_general(
        x_ref[...], w_ref[...], (((1,), (1,)), ((), ())),
        preferred_element_type=jnp.float32)


def _tc_matmul(x, W):
    return pl.pallas_call(
        _mm_body,
        grid=(_GN,),
        in_specs=[pl.BlockSpec((_BN, _D), lambda i: (i, 0)),
                  pl.BlockSpec((_D, _D), lambda i: (0, 0))],
        out_specs=pl.BlockSpec((_BN, _D), lambda i: (i, 0)),
        out_shape=jax.ShapeDtypeStruct((_N, _D), jnp.float32),
    )(x, W)


def _u0_body(y_ref, d0_ref, d1_ref, u0_ref, dv_ref):
    deg = d0_ref[...] + d1_ref[...] - 1.0
    dv = lax.rsqrt(deg)
    u0_ref[...] = dv * y_ref[...]
    dv_ref[...] = dv


def _tc_u0(y, degp):
    return pl.pallas_call(
        _u0_body,
        grid=(_GN,),
        in_specs=[pl.BlockSpec((_BN, _D), lambda i: (i, 0)),
                  pl.BlockSpec((_BN, _D), lambda i: (i, 0)),
                  pl.BlockSpec((_BN, _D), lambda i: (i + _GN, 0))],
        out_specs=[pl.BlockSpec((_BN, _D), lambda i: (i, 0)),
                   pl.BlockSpec((_BN, _D), lambda i: (i, 0))],
        out_shape=[jax.ShapeDtypeStruct((_N, _D), jnp.float32),
                   jax.ShapeDtypeStruct((_N, _D), jnp.float32)],
    )(y, degp, degp)


def _u1_body(p0_ref, p1_ref, u0_ref, dv_ref, u1_ref):
    dv = dv_ref[...]
    u1_ref[...] = dv * dv * (p0_ref[...] + p1_ref[...] - u0_ref[...])


def _tc_u1(p, u0, dvb):
    return pl.pallas_call(
        _u1_body,
        grid=(_GN,),
        in_specs=[pl.BlockSpec((_BN, _D), lambda i: (i, 0)),
                  pl.BlockSpec((_BN, _D), lambda i: (i + _GN, 0)),
                  pl.BlockSpec((_BN, _D), lambda i: (i, 0)),
                  pl.BlockSpec((_BN, _D), lambda i: (i, 0))],
        out_specs=pl.BlockSpec((_BN, _D), lambda i: (i, 0)),
        out_shape=jax.ShapeDtypeStruct((_N, _D), jnp.float32),
    )(p, p, u0, dvb)


def _fin_body(q0_ref, q1_ref, u1_ref, dv_ref, b_ref, o_ref):
    o_ref[...] = (dv_ref[...] * (q0_ref[...] + q1_ref[...] - u1_ref[...])
                  + b_ref[...])


def _tc_final(q, u1, dvb, b2d):
    return pl.pallas_call(
        _fin_body,
        grid=(_GN,),
        in_specs=[pl.BlockSpec((_BN, _D), lambda i: (i, 0)),
                  pl.BlockSpec((_BN, _D), lambda i: (i + _GN, 0)),
                  pl.BlockSpec((_BN, _D), lambda i: (i, 0)),
                  pl.BlockSpec((_BN, _D), lambda i: (i, 0)),
                  pl.BlockSpec((1, _D), lambda i: (0, 0))],
        out_specs=pl.BlockSpec((_BN, _D), lambda i: (i, 0)),
        out_shape=jax.ShapeDtypeStruct((_N, _D), jnp.float32),
    )(q, q, u1, dvb, b2d)


def kernel(x, edge_index, W, b):
    src = edge_index[0]
    dst = edge_index[1]
    ones = jnp.ones((_N, _D), jnp.float32)
    y = _tc_matmul(x, W)
    degp = _sc_degree(ones, dst)
    u0, dvb = _tc_u0(y, degp)
    p = _sc_scatter(u0, src, dst)
    u1 = _tc_u1(p, u0, dvb)
    q = _sc_scatter(u1, src, dst)
    return _tc_final(q, u1, dvb, b.reshape(1, _D))


# speedup vs baseline: 20.7330x; 1.3925x over previous
import functools

import jax
import jax.numpy as jnp
from jax import lax
from jax.experimental import pallas as pl
from jax.experimental.pallas import tpu as pltpu
from jax.experimental.pallas import tpu_sc as plsc

_N = 10000
_E = 320000
_D = 128
_NC = 2
_NS = 16
_NW = _NC * _NS
_EB = 128
_NBLK = _E // _EB
_BPT = _NBLK // _NW
_EXTRA = _NBLK - _BPT * _NW
_RS = 624
_RTAIL = _N - _RS * _NS


@functools.partial(
    pl.kernel,
    out_type=jax.ShapeDtypeStruct((2 * _N, _D), jnp.float32),
    mesh=plsc.VectorSubcoreMesh(core_axis_name="c", subcore_axis_name="s"),
    scratch_types=[
        pltpu.VMEM((_EB,), jnp.int32),
        pltpu.VMEM((_EB,), jnp.int32),
        pltpu.VMEM((_EB,), jnp.int32),
        pltpu.VMEM((_EB,), jnp.int32),
        pltpu.VMEM((_EB, _D), jnp.float32),
        pltpu.VMEM((_EB, _D), jnp.float32),
        pltpu.SemaphoreType.DMA,
        pltpu.SemaphoreType.DMA,
        pltpu.VMEM_SHARED((_N, _D), jnp.float32),
    ],
)
def _sc_scatter(u_hbm, srcb_hbm, dstb_hbm, out_hbm, sidx0, sidx1, didx0,
                didx1, rows0, rows1, gsem0, gsem1, acc_sh):
    c = lax.axis_index("c")
    s = lax.axis_index("s")
    wid = c * _NS + s
    base = wid * _BPT

    pltpu.sync_copy(u_hbm.at[pl.ds(s * _RS, _RS)],
                    acc_sh.at[pl.ds(s * _RS, _RS)])

    @pl.when(s == _NS - 1)
    def _():
        pltpu.sync_copy(u_hbm.at[pl.ds(_NS * _RS, _RTAIL)],
                        acc_sh.at[pl.ds(_NS * _RS, _RTAIL)])

    plsc.subcore_barrier()

    def _ld(b, sv, dv):
        pltpu.sync_copy(srcb_hbm.at[pl.ds(b * _EB, _EB)], sv)
        pltpu.sync_copy(dstb_hbm.at[pl.ds(b * _EB, _EB)], dv)

    @pl.when(wid < _EXTRA)
    def _():
        _ld(_NW * _BPT + wid, sidx0, didx0)
        pltpu.sync_copy(u_hbm.at[sidx0], rows0)
        pltpu.sync_copy(rows0, acc_sh.at[didx0], add=True)

    _ld(base, sidx0, didx0)
    pltpu.async_copy(u_hbm.at[sidx0], rows0, gsem0)
    _ld(base + 1, sidx1, didx1)

    @pl.loop(0, _BPT // 2)
    def _(j):
        i0 = base + 2 * j
        pltpu.make_async_copy(u_hbm.at[sidx0], rows0, gsem0).wait()
        pltpu.async_copy(u_hbm.at[sidx1], rows1, gsem1)
        pltpu.sync_copy(rows0, acc_sh.at[didx0], add=True)
        _ld(i0 + 2, sidx0, didx0)
        pltpu.make_async_copy(u_hbm.at[sidx1], rows1, gsem1).wait()
        pltpu.async_copy(u_hbm.at[sidx0], rows0, gsem0)
        pltpu.sync_copy(rows1, acc_sh.at[didx1], add=True)
        _ld(i0 + 3, sidx1, didx1)

    pltpu.make_async_copy(u_hbm.at[sidx0], rows0, gsem0).wait()

    plsc.subcore_barrier()
    pltpu.sync_copy(acc_sh.at[pl.ds(s * _RS, _RS)],
                    out_hbm.at[pl.ds(c * _N + s * _RS, _RS)])

    @pl.when(s == _NS - 1)
    def _():
        pltpu.sync_copy(acc_sh.at[pl.ds(_NS * _RS, _RTAIL)],
                        out_hbm.at[pl.ds(c * _N + _NS * _RS, _RTAIL)])


@functools.partial(
    pl.kernel,
    out_type=jax.ShapeDtypeStruct((2 * _N, _D), jnp.float32),
    mesh=plsc.VectorSubcoreMesh(core_axis_name="c", subcore_axis_name="s"),
    scratch_types=[
        pltpu.VMEM((_EB,), jnp.int32),
        pltpu.VMEM((_EB, _D), jnp.float32),
        pltpu.VMEM_SHARED((_N, _D), jnp.float32),
    ],
)
def _sc_degree(ones_hbm, dstb_hbm, out_hbm, didx_v, ones_v, acc_sh):
    c = lax.axis_index("c")
    s = lax.axis_index("s")
    wid = c * _NS + s

    pltpu.sync_copy(ones_hbm.at[pl.ds(0, _EB)], ones_v)
    pltpu.sync_copy(ones_hbm.at[pl.ds(s * _RS, _RS)],
                    acc_sh.at[pl.ds(s * _RS, _RS)])

    @pl.when(s == _NS - 1)
    def _():
        pltpu.sync_copy(ones_hbm.at[pl.ds(_NS * _RS, _RTAIL)],
                        acc_sh.at[pl.ds(_NS * _RS, _RTAIL)])

    plsc.subcore_barrier()

    def _block(b):
        pltpu.sync_copy(dstb_hbm.at[pl.ds(b * _EB, _EB)], didx_v)
        pltpu.sync_copy(ones_v, acc_sh.at[didx_v], add=True)

    @pl.loop(0, _BPT)
    def _(i):
        _block(wid * _BPT + i)

    @pl.when(wid < _EXTRA)
    def _():
        _block(_NW * _BPT + wid)

    plsc.subcore_barrier()
    pltpu.sync_copy(acc_sh.at[pl.ds(s * _RS, _RS)],
                    out_hbm.at[pl.ds(c * _N + s * _RS, _RS)])

    @pl.when(s == _NS - 1)
    def _():
        pltpu.sync_copy(acc_sh.at[pl.ds(_NS * _RS, _RTAIL)],
                        out_hbm.at[pl.ds(c * _N + _NS * _RS, _RTAIL)])


_BN = 1000
_GN = _N // _BN


def _mm_body(x_ref, w_ref, y_ref):
    y_ref[...] = lax.dot_general(
        x_ref[...], w_ref[...], (((1,), (1,)), ((), ())),
        preferred_element_type=jnp.float32)


def _tc_matmul(x, W):
    return pl.pallas_call(
        _mm_body,
        grid=(_GN,),
        in_specs=[pl.BlockSpec((_BN, _D), lambda i: (i, 0)),
                  pl.BlockSpec((_D, _D), lambda i: (0, 0))],
        out_specs=pl.BlockSpec((_BN, _D), lambda i: (i, 0)),
        out_shape=jax.ShapeDtypeStruct((_N, _D), jnp.float32),
    )(x, W)


def _u0_body(y_ref, d0_ref, d1_ref, u0_ref, dv_ref):
    deg = d0_ref[...] + d1_ref[...] - 1.0
    dv = lax.rsqrt(deg)
    u0_ref[...] = dv * y_ref[...]
    dv_ref[...] = dv


def _tc_u0(y, degp):
    return pl.pallas_call(
        _u0_body,
        grid=(_GN,),
        in_specs=[pl.BlockSpec((_BN, _D), lambda i: (i, 0)),
                  pl.BlockSpec((_BN, _D), lambda i: (i, 0)),
                  pl.BlockSpec((_BN, _D), lambda i: (i + _GN, 0))],
        out_specs=[pl.BlockSpec((_BN, _D), lambda i: (i, 0)),
                   pl.BlockSpec((_BN, _D), lambda i: (i, 0))],
        out_shape=[jax.ShapeDtypeStruct((_N, _D), jnp.float32),
                   jax.ShapeDtypeStruct((_N, _D), jnp.float32)],
    )(y, degp, degp)


def _u1_body(p0_ref, p1_ref, u0_ref, dv_ref, u1_ref):
    dv = dv_ref[...]
    u1_ref[...] = dv * dv * (p0_ref[...] + p1_ref[...] - u0_ref[...])


def _tc_u1(p, u0, dvb):
    return pl.pallas_call(
        _u1_body,
        grid=(_GN,),
        in_specs=[pl.BlockSpec((_BN, _D), lambda i: (i, 0)),
                  pl.BlockSpec((_BN, _D), lambda i: (i + _GN, 0)),
                  pl.BlockSpec((_BN, _D), lambda i: (i, 0)),
                  pl.BlockSpec((_BN, _D), lambda i: (i, 0))],
        out_specs=pl.BlockSpec((_BN, _D), lambda i: (i, 0)),
        out_shape=jax.ShapeDtypeStruct((_N, _D), jnp.float32),
    )(p, p, u0, dvb)


def _fin_body(q0_ref, q1_ref, u1_ref, dv_ref, b_ref, o_ref):
    o_ref[...] = (dv_ref[...] * (q0_ref[...] + q1_ref[...] - u1_ref[...])
                  + b_ref[...])


def _tc_final(q, u1, dvb, b2d):
    return pl.pallas_call(
        _fin_body,
        grid=(_GN,),
        in_specs=[pl.BlockSpec((_BN, _D), lambda i: (i, 0)),
                  pl.BlockSpec((_BN, _D), lambda i: (i + _GN, 0)),
                  pl.BlockSpec((_BN, _D), lambda i: (i, 0)),
                  pl.BlockSpec((_BN, _D), lambda i: (i, 0)),
                  pl.BlockSpec((1, _D), lambda i: (0, 0))],
        out_specs=pl.BlockSpec((_BN, _D), lambda i: (i, 0)),
        out_shape=jax.ShapeDtypeStruct((_N, _D), jnp.float32),
    )(q, q, u1, dvb, b2d)


def kernel(x, edge_index, W, b):
    src = edge_index[0]
    dst = edge_index[1]
    ones = jnp.ones((_N, _D), jnp.float32)
    y = _tc_matmul(x, W)
    degp = _sc_degree(ones, dst)
    u0, dvb = _tc_u0(y, degp)
    p = _sc_scatter(u0, src, dst)
    u1 = _tc_u1(p, u0, dvb)
    q = _sc_scatter(u1, src, dst)
    return _tc_final(q, u1, dvb, b.reshape(1, _D))


# speedup vs baseline: 24.3868x; 1.1762x over previous
import functools

import jax
import jax.numpy as jnp
from jax import lax
from jax.experimental import pallas as pl
from jax.experimental.pallas import tpu as pltpu
from jax.experimental.pallas import tpu_sc as plsc

_N = 10000
_E = 320000
_D = 128
_NC = 2
_NS = 16
_NW = _NC * _NS
_EB = 128
_NBLK = _E // _EB
_BPT = _NBLK // _NW
_EXTRA = _NBLK - _BPT * _NW
_RS = 624
_RTAIL = _N - _RS * _NS


@functools.partial(
    pl.kernel,
    out_type=jax.ShapeDtypeStruct((2 * _N, _D), jnp.float32),
    mesh=plsc.VectorSubcoreMesh(core_axis_name="c", subcore_axis_name="s"),
    scratch_types=[
        pltpu.VMEM((_EB,), jnp.int32),
        pltpu.VMEM((_EB,), jnp.int32),
        pltpu.VMEM((_EB,), jnp.int32),
        pltpu.VMEM((_EB,), jnp.int32),
        pltpu.VMEM((_EB, _D), jnp.float32),
        pltpu.VMEM((_EB, _D), jnp.float32),
        pltpu.SemaphoreType.DMA,
        pltpu.SemaphoreType.DMA,
        pltpu.SemaphoreType.DMA,
        pltpu.SemaphoreType.DMA,
        pltpu.VMEM_SHARED((_N, _D), jnp.float32),
    ],
)
def _sc_scatter(u_hbm, srcb_hbm, dstb_hbm, out_hbm, sidx0, sidx1, didx0,
                didx1, rows0, rows1, gsem0, gsem1, isem0, isem1, acc_sh):
    c = lax.axis_index("c")
    s = lax.axis_index("s")
    wid = c * _NS + s
    base = wid * _BPT

    pltpu.sync_copy(u_hbm.at[pl.ds(s * _RS, _RS)],
                    acc_sh.at[pl.ds(s * _RS, _RS)])

    @pl.when(s == _NS - 1)
    def _():
        pltpu.sync_copy(u_hbm.at[pl.ds(_NS * _RS, _RTAIL)],
                        acc_sh.at[pl.ds(_NS * _RS, _RTAIL)])

    plsc.subcore_barrier()

    def _lda(b, sv, dv, sem):
        pltpu.async_copy(srcb_hbm.at[pl.ds(b * _EB, _EB)], sv, sem)
        pltpu.async_copy(dstb_hbm.at[pl.ds(b * _EB, _EB)], dv, sem)

    def _wda(b, sv, dv, sem):
        pltpu.make_async_copy(srcb_hbm.at[pl.ds(b * _EB, _EB)], sv,
                              sem).wait()
        pltpu.make_async_copy(dstb_hbm.at[pl.ds(b * _EB, _EB)], dv,
                              sem).wait()

    @pl.when(wid < _EXTRA)
    def _():
        b = _NW * _BPT + wid
        _lda(b, sidx0, didx0, isem0)
        _wda(b, sidx0, didx0, isem0)
        pltpu.sync_copy(u_hbm.at[sidx0], rows0)
        pltpu.sync_copy(rows0, acc_sh.at[didx0], add=True)

    _lda(base, sidx0, didx0, isem0)
    _wda(base, sidx0, didx0, isem0)
    pltpu.async_copy(u_hbm.at[sidx0], rows0, gsem0)
    _lda(base + 1, sidx1, didx1, isem1)

    @pl.loop(0, _BPT // 2)
    def _(j):
        i0 = base + 2 * j
        pltpu.make_async_copy(u_hbm.at[sidx0], rows0, gsem0).wait()
        _wda(i0 + 1, sidx1, didx1, isem1)
        pltpu.async_copy(u_hbm.at[sidx1], rows1, gsem1)
        pltpu.sync_copy(rows0, acc_sh.at[didx0], add=True)
        _lda(i0 + 2, sidx0, didx0, isem0)
        pltpu.make_async_copy(u_hbm.at[sidx1], rows1, gsem1).wait()
        _wda(i0 + 2, sidx0, didx0, isem0)
        pltpu.async_copy(u_hbm.at[sidx0], rows0, gsem0)
        pltpu.sync_copy(rows1, acc_sh.at[didx1], add=True)
        _lda(i0 + 3, sidx1, didx1, isem1)

    pltpu.make_async_copy(u_hbm.at[sidx0], rows0, gsem0).wait()
    _wda(base + _BPT + 1, sidx1, didx1, isem1)

    plsc.subcore_barrier()
    pltpu.sync_copy(acc_sh.at[pl.ds(s * _RS, _RS)],
                    out_hbm.at[pl.ds(c * _N + s * _RS, _RS)])

    @pl.when(s == _NS - 1)
    def _():
        pltpu.sync_copy(acc_sh.at[pl.ds(_NS * _RS, _RTAIL)],
                        out_hbm.at[pl.ds(c * _N + _NS * _RS, _RTAIL)])


@functools.partial(
    pl.kernel,
    out_type=jax.ShapeDtypeStruct((2 * _N, _D), jnp.float32),
    mesh=plsc.VectorSubcoreMesh(core_axis_name="c", subcore_axis_name="s"),
    scratch_types=[
        pltpu.VMEM((_EB,), jnp.int32),
        pltpu.VMEM((_EB,), jnp.int32),
        pltpu.VMEM((_EB, _D), jnp.float32),
        pltpu.SemaphoreType.DMA,
        pltpu.SemaphoreType.DMA,
        pltpu.VMEM_SHARED((_N, _D), jnp.float32),
    ],
)
def _sc_degree(ones_hbm, dstb_hbm, out_hbm, didx0, didx1, ones_v, isem0,
               isem1, acc_sh):
    c = lax.axis_index("c")
    s = lax.axis_index("s")
    wid = c * _NS + s
    base = wid * _BPT

    pltpu.sync_copy(ones_hbm.at[pl.ds(0, _EB)], ones_v)
    pltpu.sync_copy(ones_hbm.at[pl.ds(s * _RS, _RS)],
                    acc_sh.at[pl.ds(s * _RS, _RS)])

    @pl.when(s == _NS - 1)
    def _():
        pltpu.sync_copy(ones_hbm.at[pl.ds(_NS * _RS, _RTAIL)],
                        acc_sh.at[pl.ds(_NS * _RS, _RTAIL)])

    plsc.subcore_barrier()

    def _ld(b, dv, sem):
        pltpu.async_copy(dstb_hbm.at[pl.ds(b * _EB, _EB)], dv, sem)

    def _wd(b, dv, sem):
        pltpu.make_async_copy(dstb_hbm.at[pl.ds(b * _EB, _EB)], dv,
                              sem).wait()

    @pl.when(wid < _EXTRA)
    def _():
        b = _NW * _BPT + wid
        _ld(b, didx0, isem0)
        _wd(b, didx0, isem0)
        pltpu.sync_copy(ones_v, acc_sh.at[didx0], add=True)

    _ld(base, didx0, isem0)
    _ld(base + 1, didx1, isem1)

    @pl.loop(0, _BPT // 2)
    def _(j):
        i0 = base + 2 * j
        _wd(i0, didx0, isem0)
        pltpu.sync_copy(ones_v, acc_sh.at[didx0], add=True)
        _ld(i0 + 2, didx0, isem0)
        _wd(i0 + 1, didx1, isem1)
        pltpu.sync_copy(ones_v, acc_sh.at[didx1], add=True)
        _ld(i0 + 3, didx1, isem1)

    _wd(base + _BPT, didx0, isem0)
    _wd(base + _BPT + 1, didx1, isem1)

    plsc.subcore_barrier()
    pltpu.sync_copy(acc_sh.at[pl.ds(s * _RS, _RS)],
                    out_hbm.at[pl.ds(c * _N + s * _RS, _RS)])

    @pl.when(s == _NS - 1)
    def _():
        pltpu.sync_copy(acc_sh.at[pl.ds(_NS * _RS, _RTAIL)],
                        out_hbm.at[pl.ds(c * _N + _NS * _RS, _RTAIL)])


_BN = 1000
_GN = _N // _BN


def _mm_body(x_ref, w_ref, y_ref):
    y_ref[...] = lax.dot_general(
        x_ref[...], w_ref[...], (((1,), (1,)), ((), ())),
        preferred_element_type=jnp.float32)


def _tc_matmul(x, W):
    return pl.pallas_call(
        _mm_body,
        grid=(_GN,),
        in_specs=[pl.BlockSpec((_BN, _D), lambda i: (i, 0)),
                  pl.BlockSpec((_D, _D), lambda i: (0, 0))],
        out_specs=pl.BlockSpec((_BN, _D), lambda i: (i, 0)),
        out_shape=jax.ShapeDtypeStruct((_N, _D), jnp.float32),
    )(x, W)


def _u0_body(y_ref, d0_ref, d1_ref, u0_ref, dv_ref):
    deg = d0_ref[...] + d1_ref[...] - 1.0
    dv = lax.rsqrt(deg)
    u0_ref[...] = dv * y_ref[...]
    dv_ref[...] = dv


def _tc_u0(y, degp):
    return pl.pallas_call(
        _u0_body,
        grid=(_GN,),
        in_specs=[pl.BlockSpec((_BN, _D), lambda i: (i, 0)),
                  pl.BlockSpec((_BN, _D), lambda i: (i, 0)),
                  pl.BlockSpec((_BN, _D), lambda i: (i + _GN, 0))],
        out_specs=[pl.BlockSpec((_BN, _D), lambda i: (i, 0)),
                   pl.BlockSpec((_BN, _D), lambda i: (i, 0))],
        out_shape=[jax.ShapeDtypeStruct((_N, _D), jnp.float32),
                   jax.ShapeDtypeStruct((_N, _D), jnp.float32)],
    )(y, degp, degp)


def _u1_body(p0_ref, p1_ref, u0_ref, dv_ref, u1_ref):
    dv = dv_ref[...]
    u1_ref[...] = dv * dv * (p0_ref[...] + p1_ref[...] - u0_ref[...])


def _tc_u1(p, u0, dvb):
    return pl.pallas_call(
        _u1_body,
        grid=(_GN,),
        in_specs=[pl.BlockSpec((_BN, _D), lambda i: (i, 0)),
                  pl.BlockSpec((_BN, _D), lambda i: (i + _GN, 0)),
                  pl.BlockSpec((_BN, _D), lambda i: (i, 0)),
                  pl.BlockSpec((_BN, _D), lambda i: (i, 0))],
        out_specs=pl.BlockSpec((_BN, _D), lambda i: (i, 0)),
        out_shape=jax.ShapeDtypeStruct((_N, _D), jnp.float32),
    )(p, p, u0, dvb)


def _fin_body(q0_ref, q1_ref, u1_ref, dv_ref, b_ref, o_ref):
    o_ref[...] = (dv_ref[...] * (q0_ref[...] + q1_ref[...] - u1_ref[...])
                  + b_ref[...])


def _tc_final(q, u1, dvb, b2d):
    return pl.pallas_call(
        _fin_body,
        grid=(_GN,),
        in_specs=[pl.BlockSpec((_BN, _D), lambda i: (i, 0)),
                  pl.BlockSpec((_BN, _D), lambda i: (i + _GN, 0)),
                  pl.BlockSpec((_BN, _D), lambda i: (i, 0)),
                  pl.BlockSpec((_BN, _D), lambda i: (i, 0)),
                  pl.BlockSpec((1, _D), lambda i: (0, 0))],
        out_specs=pl.BlockSpec((_BN, _D), lambda i: (i, 0)),
        out_shape=jax.ShapeDtypeStruct((_N, _D), jnp.float32),
    )(q, q, u1, dvb, b2d)


def kernel(x, edge_index, W, b):
    src = edge_index[0]
    dst = edge_index[1]
    ones = jnp.ones((_N, _D), jnp.float32)
    y = _tc_matmul(x, W)
    degp = _sc_degree(ones, dst)
    u0, dvb = _tc_u0(y, degp)
    p = _sc_scatter(u0, src, dst)
    u1 = _tc_u1(p, u0, dvb)
    q = _sc_scatter(u1, src, dst)
    return _tc_final(q, u1, dvb, b.reshape(1, _D))
